# Initial kernel scaffold; baseline (speedup 1.0000x reference)
import jax
import jax.numpy as jnp
from jax.experimental import pallas as pl


def kernel(in_nodes_features, edge_index, W_proj, a_src, a_tgt, W_skip, bias):
    raise NotImplementedError("write your pallas kernel here")


# speedup vs baseline: 21.4697x; 21.4697x over previous
import dataclasses

import jax
import jax.numpy as jnp
from jax import lax
from jax.experimental import pallas as pl
from jax.experimental.pallas import tpu as pltpu
from jax.experimental.pallas import tpu_sc as plsc

N = 10000
E = 160000
D = 256
H = 8
F = 32
HF = H * F
HP = 16

NC = 2
NS = 16
NW = NC * NS
EPW = E // NW
NPT = N // NS

B1 = 1000
R1 = 400


def _sc_compiler_params():
    cp = pltpu.CompilerParams()
    fields = pltpu.CompilerParams.__dataclass_fields__
    if "needs_layout_passes" in fields:
        cp = dataclasses.replace(cp, needs_layout_passes=False)
    if "use_tc_tiling_on_sc" in fields:
        cp = dataclasses.replace(cp, use_tc_tiling_on_sc=False)
    return cp


def _mesh():
    return plsc.VectorSubcoreMesh(core_axis_name="c", subcore_axis_name="s")


def _tc1_body(x_ref, wpt_ref, wst_ref, asrc_ref, atgt_ref,
              proj_ref, ssrc_ref, stgt_ref, skip_ref):
    x = x_ref[...]
    p = jnp.dot(x, wpt_ref[...], preferred_element_type=jnp.float32)
    skip_ref[...] = jnp.dot(x, wst_ref[...], preferred_element_type=jnp.float32)
    ssrc_ref[...] = jnp.dot(p, asrc_ref[...], preferred_element_type=jnp.float32)
    stgt_ref[...] = jnp.dot(p, atgt_ref[...], preferred_element_type=jnp.float32)
    for h in range(H):
        proj_ref[h] = p[:, h * F:(h + 1) * F]


def _tc1(x, wpt, wst, amat_src, amat_tgt):
    return pl.pallas_call(
        _tc1_body,
        grid=(N // R1,),
        in_specs=[
            pl.BlockSpec((R1, D), lambda i: (i, 0)),
            pl.BlockSpec((D, HF), lambda i: (0, 0)),
            pl.BlockSpec((D, HF), lambda i: (0, 0)),
            pl.BlockSpec((HF, HP), lambda i: (0, 0)),
            pl.BlockSpec((HF, HP), lambda i: (0, 0)),
        ],
        out_specs=[
            pl.BlockSpec((H, R1, F), lambda i: (0, i, 0)),
            pl.BlockSpec((R1, HP), lambda i: (i, 0)),
            pl.BlockSpec((R1, HP), lambda i: (i, 0)),
            pl.BlockSpec((R1, HF), lambda i: (i, 0)),
        ],
        out_shape=[
            jax.ShapeDtypeStruct((H, N, F), jnp.float32),
            jax.ShapeDtypeStruct((N, HP), jnp.float32),
            jax.ShapeDtypeStruct((N, HP), jnp.float32),
            jax.ShapeDtypeStruct((N, HF), jnp.float32),
        ],
    )(x, wpt, wst, amat_src, amat_tgt)


def _sc1_body(ssrc_hbm, stgt_hbm, src_hbm, dst_hbm,
              exprows_hbm, den_hbm,
              srcv, dstv, g1, g2, ev, zv, den_sh):
    cid = lax.axis_index("c")
    sid = lax.axis_index("s")

    @pl.loop(0, NPT)
    def _(i):
        zv[i] = jnp.zeros((HP,), jnp.float32)

    pltpu.sync_copy(zv, den_sh.at[pl.ds(sid * NPT, NPT)])
    plsc.subcore_barrier()

    base = (sid * NC + cid) * EPW
    for blk in range(EPW // B1):
        eb = base + blk * B1
        pltpu.sync_copy(src_hbm.at[pl.ds(eb, B1)], srcv)
        pltpu.sync_copy(dst_hbm.at[pl.ds(eb, B1)], dstv)
        pltpu.sync_copy(ssrc_hbm.at[srcv], g1)
        pltpu.sync_copy(stgt_hbm.at[dstv], g2)

        @pl.loop(0, B1)
        def _(j):
            s = g1[j] + g2[j]
            s = jnp.where(s > 0.0, s, 0.2 * s)
            ev[j] = jnp.exp(s)

        pltpu.sync_copy(ev, den_sh.at[dstv], add=True)
        pltpu.sync_copy(ev, exprows_hbm.at[pl.ds(eb, B1)])

    plsc.subcore_barrier()
    r0 = sid * 624
    pltpu.sync_copy(den_sh.at[pl.ds(r0, 624)], den_hbm.at[cid, pl.ds(r0, 624)])

    @pl.when(sid == NS - 1)
    def _():
        pltpu.sync_copy(den_sh.at[pl.ds(624 * NS, 16)],
                        den_hbm.at[cid, pl.ds(624 * NS, 16)])


def _sc1(ssrc, stgt, src, dst):
    f = pl.kernel(
        _sc1_body,
        out_type=(jax.ShapeDtypeStruct((E, HP), jnp.float32),
                  jax.ShapeDtypeStruct((NC, N, HP), jnp.float32)),
        mesh=_mesh(),
        scratch_types=[
            pltpu.VMEM((B1,), jnp.int32),
            pltpu.VMEM((B1,), jnp.int32),
            pltpu.VMEM((B1, HP), jnp.float32),
            pltpu.VMEM((B1, HP), jnp.float32),
            pltpu.VMEM((B1, HP), jnp.float32),
            pltpu.VMEM((NPT, HP), jnp.float32),
            pltpu.VMEM_SHARED((N, HP), jnp.float32),
        ],
        compiler_params=_sc_compiler_params(),
    )
    return f(ssrc, stgt, src, dst)


def _tc2_body(den_ref, rec_ref):
    rec_ref[...] = 1.0 / (den_ref[0] + den_ref[1] + 1e-16)


def _tc2(den):
    return pl.pallas_call(
        _tc2_body,
        grid=(N // 2000,),
        in_specs=[pl.BlockSpec((NC, 2000, HP), lambda i: (0, i, 0))],
        out_specs=pl.BlockSpec((2000, HP), lambda i: (i, 0)),
        out_shape=jax.ShapeDtypeStruct((N, HP), jnp.float32),
    )(den)


def _sca_body(exprows_hbm, recip_hbm, dst_hbm, attt_hbm,
              dstv, ev, rv, atv):
    cid = lax.axis_index("c")
    sid = lax.axis_index("s")
    lanes = jnp.arange(HP, dtype=jnp.int32)

    base = (sid * NC + cid) * EPW
    for blk in range(EPW // B1):
        eb = base + blk * B1
        pltpu.sync_copy(dst_hbm.at[pl.ds(eb, B1)], dstv)
        pltpu.sync_copy(exprows_hbm.at[pl.ds(eb, B1)], ev)
        pltpu.sync_copy(recip_hbm.at[dstv], rv)

        @pl.loop(0, B1)
        def _(j):
            a = ev[j] * rv[j]
            plsc.store_scatter(atv, [lanes, jnp.full((HP,), j, jnp.int32)], a)

        for hh in range(HP):
            pltpu.sync_copy(atv.at[hh], attt_hbm.at[pl.ds(hh * E + eb, B1)])


def _sca(exprows, recip, dst):
    f = pl.kernel(
        _sca_body,
        out_type=jax.ShapeDtypeStruct((HP * E,), jnp.float32),
        mesh=_mesh(),
        scratch_types=[
            pltpu.VMEM((B1,), jnp.int32),
            pltpu.VMEM((B1, HP), jnp.float32),
            pltpu.VMEM((B1, HP), jnp.float32),
            pltpu.VMEM((HP, B1), jnp.float32),
        ],
        compiler_params=_sc_compiler_params(),
    )
    return f(exprows, recip, dst)


def _sc3_body(p0, p1, p2, p3, p4, p5, p6, p7, attt_hbm,
              src_hbm, dst_hbm, outp_hbm,
              srcv, dstv, rows, attv, wbuf, zv2, out_sh):
    proj_refs = [p0, p1, p2, p3, p4, p5, p6, p7]
    cid = lax.axis_index("c")
    sid = lax.axis_index("s")

    @pl.loop(0, NPT)
    def _(i):
        zv2[i, pl.ds(0, 16)] = jnp.zeros((16,), jnp.float32)
        zv2[i, pl.ds(16, 16)] = jnp.zeros((16,), jnp.float32)

    ebase = cid * (E // NC) + sid * EPW
    for h in range(H):
        pltpu.sync_copy(zv2, out_sh.at[pl.ds(sid * NPT, NPT)])
        plsc.subcore_barrier()

        for blk in range(EPW // B1):
            eb = ebase + blk * B1
            pltpu.sync_copy(src_hbm.at[pl.ds(eb, B1)], srcv)
            pltpu.sync_copy(dst_hbm.at[pl.ds(eb, B1)], dstv)
            pltpu.sync_copy(proj_refs[h].at[srcv], rows)
            pltpu.sync_copy(attt_hbm.at[pl.ds(h * E + eb, B1)], attv)

            @pl.loop(0, B1)
            def _(b):
                av = plsc.load_gather(attv, [jnp.full((16,), b, jnp.int32)])
                wbuf[b, pl.ds(0, 16)] = rows[b, pl.ds(0, 16)] * av
                wbuf[b, pl.ds(16, 16)] = rows[b, pl.ds(16, 16)] * av

            pltpu.sync_copy(wbuf, out_sh.at[dstv], add=True)

        plsc.subcore_barrier()
        pltpu.sync_copy(out_sh.at[pl.ds(sid * 624, 624)],
                        outp_hbm.at[cid, h, pl.ds(sid * 624, 624)])

        @pl.when(sid == NS - 1)
        def _():
            pltpu.sync_copy(out_sh.at[pl.ds(624 * NS, 16)],
                            outp_hbm.at[cid, h, pl.ds(624 * NS, 16)])

        plsc.subcore_barrier()


def _sc3(proj_heads, attt, src, dst):
    f = pl.kernel(
        _sc3_body,
        out_type=jax.ShapeDtypeStruct((NC, H, N, F), jnp.float32),
        mesh=_mesh(),
        scratch_types=[
            pltpu.VMEM((B1,), jnp.int32),
            pltpu.VMEM((B1,), jnp.int32),
            pltpu.VMEM((B1, F), jnp.float32),
            pltpu.VMEM((B1,), jnp.float32),
            pltpu.VMEM((B1, F), jnp.float32),
            pltpu.VMEM((NPT, F), jnp.float32),
            pltpu.VMEM_SHARED((N, F), jnp.float32),
        ],
        compiler_params=_sc_compiler_params(),
    )
    return f(*proj_heads, attt, src, dst)


def _tc3_body(parts_ref, skip_ref, bias_ref, out_ref):
    s = parts_ref[0] + parts_ref[1]
    cat = jnp.concatenate([s[h] for h in range(H)], axis=-1)
    t = cat + skip_ref[...] + bias_ref[...]
    out_ref[...] = jnp.where(t > 0.0, t, jnp.exp(t) - 1.0)


def _tc3(parts, skip, bias2d):
    return pl.pallas_call(
        _tc3_body,
        grid=(N // R1,),
        in_specs=[
            pl.BlockSpec((NC, H, R1, F), lambda i: (0, 0, i, 0)),
            pl.BlockSpec((R1, HF), lambda i: (i, 0)),
            pl.BlockSpec((1, HF), lambda i: (0, 0)),
        ],
        out_specs=pl.BlockSpec((R1, HF), lambda i: (i, 0)),
        out_shape=jax.ShapeDtypeStruct((N, HF), jnp.float32),
    )(parts, skip, bias2d)


def _amat(a):
    z = jnp.zeros((H, F, HP), jnp.float32)
    z = z.at[jnp.arange(H), :, jnp.arange(H)].set(a[0])
    return z.reshape(HF, HP)


def kernel(in_nodes_features, edge_index, W_proj, a_src, a_tgt, W_skip, bias):
    x = in_nodes_features
    src = edge_index[0].astype(jnp.int32)
    dst = edge_index[1].astype(jnp.int32)
    proj_t, ssrc, stgt, skip = _tc1(x, W_proj.T, W_skip.T,
                                    _amat(a_src), _amat(a_tgt))
    exprows, den = _sc1(ssrc, stgt, src, dst)
    recip = _tc2(den)
    attt = _sca(exprows, recip, dst)
    parts = _sc3([proj_t[h] for h in range(H)], attt, src, dst)
    return _tc3(parts, skip, bias.reshape(1, HF))


# speedup vs baseline: 34.4872x; 1.6063x over previous
import dataclasses

import jax
import jax.numpy as jnp
from jax import lax
from jax.experimental import pallas as pl
from jax.experimental.pallas import tpu as pltpu
from jax.experimental.pallas import tpu_sc as plsc

N = 10000
E = 160000
D = 256
H = 8
F = 32
HF = H * F
HP = 16

NC = 2
NS = 16
NW = NC * NS
EPW = E // NW
NPT = N // NS

B1 = 1000
R1 = 400


def _sc_compiler_params():
    cp = pltpu.CompilerParams()
    fields = pltpu.CompilerParams.__dataclass_fields__
    if "needs_layout_passes" in fields:
        cp = dataclasses.replace(cp, needs_layout_passes=False)
    if "use_tc_tiling_on_sc" in fields:
        cp = dataclasses.replace(cp, use_tc_tiling_on_sc=False)
    return cp


def _mesh():
    return plsc.VectorSubcoreMesh(core_axis_name="c", subcore_axis_name="s")


def _tc1_body(x_ref, wpt_ref, wst_ref, asrc_ref, atgt_ref,
              proj_ref, ssrc_ref, stgt_ref, skip_ref):
    x = x_ref[...]
    p = jnp.dot(x, wpt_ref[...], preferred_element_type=jnp.float32)
    skip_ref[...] = jnp.dot(x, wst_ref[...], preferred_element_type=jnp.float32)
    ssrc_ref[...] = jnp.dot(p, asrc_ref[...], preferred_element_type=jnp.float32)
    stgt_ref[...] = jnp.dot(p, atgt_ref[...], preferred_element_type=jnp.float32)
    for h in range(H):
        proj_ref[h] = p[:, h * F:(h + 1) * F]


def _tc1(x, wpt, wst, amat_src, amat_tgt):
    return pl.pallas_call(
        _tc1_body,
        grid=(N // R1,),
        in_specs=[
            pl.BlockSpec((R1, D), lambda i: (i, 0)),
            pl.BlockSpec((D, HF), lambda i: (0, 0)),
            pl.BlockSpec((D, HF), lambda i: (0, 0)),
            pl.BlockSpec((HF, HP), lambda i: (0, 0)),
            pl.BlockSpec((HF, HP), lambda i: (0, 0)),
        ],
        out_specs=[
            pl.BlockSpec((H, R1, F), lambda i: (0, i, 0)),
            pl.BlockSpec((R1, HP), lambda i: (i, 0)),
            pl.BlockSpec((R1, HP), lambda i: (i, 0)),
            pl.BlockSpec((R1, HF), lambda i: (i, 0)),
        ],
        out_shape=[
            jax.ShapeDtypeStruct((H, N, F), jnp.float32),
            jax.ShapeDtypeStruct((N, HP), jnp.float32),
            jax.ShapeDtypeStruct((N, HP), jnp.float32),
            jax.ShapeDtypeStruct((N, HF), jnp.float32),
        ],
    )(x, wpt, wst, amat_src, amat_tgt)


def _sc1_body(ssrc_hbm, stgt_hbm, src_hbm, dst_hbm,
              exprows_hbm, den_hbm,
              srcv, dstv, g1, g2, ev, zv, den_sh):
    cid = lax.axis_index("c")
    sid = lax.axis_index("s")

    @pl.loop(0, NPT)
    def _(i):
        zv[i] = jnp.zeros((HP,), jnp.float32)

    pltpu.sync_copy(zv, den_sh.at[pl.ds(sid * NPT, NPT)])
    plsc.subcore_barrier()

    base = (sid * NC + cid) * EPW
    for blk in range(EPW // B1):
        eb = base + blk * B1
        pltpu.sync_copy(src_hbm.at[pl.ds(eb, B1)], srcv)
        pltpu.sync_copy(dst_hbm.at[pl.ds(eb, B1)], dstv)
        pltpu.sync_copy(ssrc_hbm.at[srcv], g1)
        pltpu.sync_copy(stgt_hbm.at[dstv], g2)

        @plsc.parallel_loop(0, B1, unroll=4)
        def _(j):
            s = g1[j] + g2[j]
            s = jnp.where(s > 0.0, s, 0.2 * s)
            ev[j] = jnp.exp(s)

        pltpu.sync_copy(ev, den_sh.at[dstv], add=True)
        pltpu.sync_copy(ev, exprows_hbm.at[pl.ds(eb, B1)])

    plsc.subcore_barrier()
    r0 = sid * 624
    pltpu.sync_copy(den_sh.at[pl.ds(r0, 624)], den_hbm.at[cid, pl.ds(r0, 624)])

    @pl.when(sid == NS - 1)
    def _():
        pltpu.sync_copy(den_sh.at[pl.ds(624 * NS, 16)],
                        den_hbm.at[cid, pl.ds(624 * NS, 16)])


def _sc1(ssrc, stgt, src, dst):
    f = pl.kernel(
        _sc1_body,
        out_type=(jax.ShapeDtypeStruct((E, HP), jnp.float32),
                  jax.ShapeDtypeStruct((NC, N, HP), jnp.float32)),
        mesh=_mesh(),
        scratch_types=[
            pltpu.VMEM((B1,), jnp.int32),
            pltpu.VMEM((B1,), jnp.int32),
            pltpu.VMEM((B1, HP), jnp.float32),
            pltpu.VMEM((B1, HP), jnp.float32),
            pltpu.VMEM((B1, HP), jnp.float32),
            pltpu.VMEM((NPT, HP), jnp.float32),
            pltpu.VMEM_SHARED((N, HP), jnp.float32),
        ],
        compiler_params=_sc_compiler_params(),
    )
    return f(ssrc, stgt, src, dst)


def _tc2_body(den_ref, rec_ref):
    rec_ref[...] = 1.0 / (den_ref[0] + den_ref[1] + 1e-16)


def _tc2(den):
    return pl.pallas_call(
        _tc2_body,
        grid=(N // 2000,),
        in_specs=[pl.BlockSpec((NC, 2000, HP), lambda i: (0, i, 0))],
        out_specs=pl.BlockSpec((2000, HP), lambda i: (i, 0)),
        out_shape=jax.ShapeDtypeStruct((N, HP), jnp.float32),
    )(den)


def _sca_body(exprows_hbm, recip_hbm, dst_hbm, attt_hbm,
              dstv, ev, rv, atv):
    cid = lax.axis_index("c")
    sid = lax.axis_index("s")
    lanes = jnp.arange(HP, dtype=jnp.int32)

    base = (sid * NC + cid) * EPW
    for blk in range(EPW // B1):
        eb = base + blk * B1
        pltpu.sync_copy(dst_hbm.at[pl.ds(eb, B1)], dstv)
        pltpu.sync_copy(exprows_hbm.at[pl.ds(eb, B1)], ev)
        pltpu.sync_copy(recip_hbm.at[dstv], rv)

        @plsc.parallel_loop(0, B1, unroll=4)
        def _(j):
            a = ev[j] * rv[j]
            plsc.store_scatter(atv, [lanes, jnp.full((HP,), j, jnp.int32)], a)

        for hh in range(HP):
            pltpu.sync_copy(atv.at[hh], attt_hbm.at[pl.ds(hh * E + eb, B1)])


def _sca(exprows, recip, dst):
    f = pl.kernel(
        _sca_body,
        out_type=jax.ShapeDtypeStruct((HP * E,), jnp.float32),
        mesh=_mesh(),
        scratch_types=[
            pltpu.VMEM((B1,), jnp.int32),
            pltpu.VMEM((B1, HP), jnp.float32),
            pltpu.VMEM((B1, HP), jnp.float32),
            pltpu.VMEM((HP, B1), jnp.float32),
        ],
        compiler_params=_sc_compiler_params(),
    )
    return f(exprows, recip, dst)


def _sc3_body(p0, p1, p2, p3, p4, p5, p6, p7, attt_hbm,
              src_hbm, dst_hbm, outp_hbm,
              srcv, dstv, rows, attv, wbuf, zv2, out_sh):
    proj_refs = [p0, p1, p2, p3, p4, p5, p6, p7]
    cid = lax.axis_index("c")
    sid = lax.axis_index("s")

    @pl.loop(0, NPT)
    def _(i):
        zv2[i, pl.ds(0, 16)] = jnp.zeros((16,), jnp.float32)
        zv2[i, pl.ds(16, 16)] = jnp.zeros((16,), jnp.float32)

    ebase = cid * (E // NC) + sid * EPW
    for h in range(H):
        pltpu.sync_copy(zv2, out_sh.at[pl.ds(sid * NPT, NPT)])
        plsc.subcore_barrier()

        for blk in range(EPW // B1):
            eb = ebase + blk * B1
            pltpu.sync_copy(src_hbm.at[pl.ds(eb, B1)], srcv)
            pltpu.sync_copy(dst_hbm.at[pl.ds(eb, B1)], dstv)
            pltpu.sync_copy(proj_refs[h].at[srcv], rows)
            pltpu.sync_copy(attt_hbm.at[pl.ds(h * E + eb, B1)], attv)

            @plsc.parallel_loop(0, B1, unroll=4)
            def _(b):
                av = plsc.load_gather(attv, [jnp.full((16,), b, jnp.int32)])
                wbuf[b, pl.ds(0, 16)] = rows[b, pl.ds(0, 16)] * av
                wbuf[b, pl.ds(16, 16)] = rows[b, pl.ds(16, 16)] * av

            pltpu.sync_copy(wbuf, out_sh.at[dstv], add=True)

        plsc.subcore_barrier()
        pltpu.sync_copy(out_sh.at[pl.ds(sid * 624, 624)],
                        outp_hbm.at[cid, h, pl.ds(sid * 624, 624)])

        @pl.when(sid == NS - 1)
        def _():
            pltpu.sync_copy(out_sh.at[pl.ds(624 * NS, 16)],
                            outp_hbm.at[cid, h, pl.ds(624 * NS, 16)])

        plsc.subcore_barrier()


def _sc3(proj_heads, attt, src, dst):
    f = pl.kernel(
        _sc3_body,
        out_type=jax.ShapeDtypeStruct((NC, H, N, F), jnp.float32),
        mesh=_mesh(),
        scratch_types=[
            pltpu.VMEM((B1,), jnp.int32),
            pltpu.VMEM((B1,), jnp.int32),
            pltpu.VMEM((B1, F), jnp.float32),
            pltpu.VMEM((B1,), jnp.float32),
            pltpu.VMEM((B1, F), jnp.float32),
            pltpu.VMEM((NPT, F), jnp.float32),
            pltpu.VMEM_SHARED((N, F), jnp.float32),
        ],
        compiler_params=_sc_compiler_params(),
    )
    return f(*proj_heads, attt, src, dst)


def _tc3_body(parts_ref, skip_ref, bias_ref, out_ref):
    s = parts_ref[0] + parts_ref[1]
    cat = jnp.concatenate([s[h] for h in range(H)], axis=-1)
    t = cat + skip_ref[...] + bias_ref[...]
    out_ref[...] = jnp.where(t > 0.0, t, jnp.exp(t) - 1.0)


def _tc3(parts, skip, bias2d):
    return pl.pallas_call(
        _tc3_body,
        grid=(N // R1,),
        in_specs=[
            pl.BlockSpec((NC, H, R1, F), lambda i: (0, 0, i, 0)),
            pl.BlockSpec((R1, HF), lambda i: (i, 0)),
            pl.BlockSpec((1, HF), lambda i: (0, 0)),
        ],
        out_specs=pl.BlockSpec((R1, HF), lambda i: (i, 0)),
        out_shape=jax.ShapeDtypeStruct((N, HF), jnp.float32),
    )(parts, skip, bias2d)


def _amat(a):
    z = jnp.zeros((H, F, HP), jnp.float32)
    z = z.at[jnp.arange(H), :, jnp.arange(H)].set(a[0])
    return z.reshape(HF, HP)


def kernel(in_nodes_features, edge_index, W_proj, a_src, a_tgt, W_skip, bias):
    x = in_nodes_features
    src = edge_index[0].astype(jnp.int32)
    dst = edge_index[1].astype(jnp.int32)
    proj_t, ssrc, stgt, skip = _tc1(x, W_proj.T, W_skip.T,
                                    _amat(a_src), _amat(a_tgt))
    exprows, den = _sc1(ssrc, stgt, src, dst)
    recip = _tc2(den)
    attt = _sca(exprows, recip, dst)
    parts = _sc3([proj_t[h] for h in range(H)], attt, src, dst)
    return _tc3(parts, skip, bias.reshape(1, HF))
